# baseline (speedup 1.0000x reference)
import functools

import jax
import jax.numpy as jnp
from jax import lax
from jax.experimental import pallas as pl
from jax.experimental.pallas import tpu as pltpu
from jax.experimental.pallas import tpu_sc as plsc

_B = 16384
_DIM = 64
_NC = 2
_NS = 16
_LANES = 16
_NW = _NC * _NS
_BPW = _B // _NW
_GCH = 128


def _permute16(x, idx):
    dn = lax.GatherDimensionNumbers(
        offset_dims=(), collapsed_slice_dims=(0,), start_index_map=(0,))
    return lax.gather(x, idx[:, None], dn, (1,),
                      mode=lax.GatherScatterMode.PROMISE_IN_BOUNDS)


def _sqrt16(x):
    y = jnp.where(x > 4096.0, 0.001 * x + 64.0, 0.0625 * x + 4.0)
    for _ in range(7):
        y = 0.5 * (y + x / y)
    return jnp.where(x > 0.0, y, 0.0)


def _body(heads_hbm, rels_hbm, tails_hbm, ent_hbm, relt_hbm, out_hbm,
          hidx, ridx, tidx, h_rows, r_rows, t_rows, scores_v, sem):
    wid = lax.axis_index("s") * _NC + lax.axis_index("c")
    base = wid * _BPW

    pltpu.sync_copy(heads_hbm.at[pl.ds(base, _BPW)], hidx)
    pltpu.sync_copy(rels_hbm.at[pl.ds(base, _BPW)], ridx)
    pltpu.sync_copy(tails_hbm.at[pl.ds(base, _BPW)], tidx)

    handles = []
    for j in range(_BPW // _GCH):
        sl = pl.ds(j * _GCH, _GCH)
        handles.append(pltpu.async_copy(ent_hbm.at[hidx.at[sl]], h_rows.at[sl], sem))
        handles.append(pltpu.async_copy(ent_hbm.at[tidx.at[sl]], t_rows.at[sl], sem))
        handles.append(pltpu.async_copy(relt_hbm.at[ridx.at[sl]], r_rows.at[sl], sem))
    for h in handles:
        h.wait()

    lanes = jnp.arange(_LANES, dtype=jnp.int32)
    perms = [lanes ^ (1 << p) for p in range(4)]

    def grp_body(g, carry):
        res = jnp.zeros((_LANES,), jnp.float32)
        for l in range(_LANES):
            i = g * _LANES + l
            acc = jnp.zeros((_LANES,), jnp.float32)
            for j in range(_DIM // _LANES):
                sl = pl.ds(j * _LANES, _LANES)
                d = (h_rows[i, sl] + r_rows[i, sl]) - t_rows[i, sl]
                acc = acc + d * d
            for p in perms:
                acc = acc + _permute16(acc, p)
            res = jnp.where(lanes == l, acc, res)
        scores_v[pl.ds(g * _LANES, _LANES)] = _sqrt16(res)
        return carry

    lax.fori_loop(0, _BPW // _LANES, grp_body, 0)

    pltpu.sync_copy(scores_v, out_hbm.at[pl.ds(base, _BPW)])


_mesh = plsc.VectorSubcoreMesh(core_axis_name="c", subcore_axis_name="s")

_kernel_call = pl.kernel(
    _body,
    out_type=jax.ShapeDtypeStruct((_B,), jnp.float32),
    scratch_types=[
        pltpu.VMEM((_BPW,), jnp.int32),
        pltpu.VMEM((_BPW,), jnp.int32),
        pltpu.VMEM((_BPW,), jnp.int32),
        pltpu.VMEM((_BPW, _DIM), jnp.float32),
        pltpu.VMEM((_BPW, _DIM), jnp.float32),
        pltpu.VMEM((_BPW, _DIM), jnp.float32),
        pltpu.VMEM((_BPW,), jnp.float32),
        pltpu.SemaphoreType.DMA,
    ],
    mesh=_mesh,
    compiler_params=pltpu.CompilerParams(use_tc_tiling_on_sc=False),
)


@jax.jit
def kernel(heads, rels, tails, entity_table, rel_table):
    return _kernel_call(heads, rels, tails, entity_table, rel_table)
